# baseline (speedup 1.0000x reference)
import functools

import jax
import jax.numpy as jnp
from jax import lax
from jax.experimental import pallas as pl
from jax.experimental.pallas import tpu as pltpu
from jax.experimental.pallas import tpu_sc as plsc

_B, _N, _D, _R, _K = 4, 4096, 128, 256, 64
_NEG = -3.0e38
_NC, _NS = 2, 16
_NW = _NC * _NS
_RPW = (_B * _K) // _NW
_RC = 4


def _phase1a_body(ne, ht, wmap, bmap,
                  g64_out, vals_out, scores_out, ridx_out, cidx_out,
                  scorer_out):
    f32 = jnp.float32
    nt = (((1,), (1,)), ((), ()))

    htr = ht[...]
    scorer = jnp.tanh(
        lax.dot_general(htr, wmap[...], nt, preferred_element_type=f32)
        + bmap[...])
    snorm = jnp.sqrt(jnp.sum(scorer * scorer, axis=1, keepdims=True))

    _H = _N // 2
    rows = [None] * (2 * _B)
    for b in range(_B):
        rows[b] = lax.dot_general(scorer[b:b + 1, :], ne[b, :_H],
                                  nt, preferred_element_type=f32)
        rows[_B + b] = lax.dot_general(scorer[b:b + 1, :], ne[b, _H:],
                                       nt, preferred_element_type=f32)
    snorm8 = jnp.concatenate([snorm, snorm], axis=0)
    scores = jnp.concatenate(rows, axis=0) / snorm8

    iota_row = lax.broadcasted_iota(jnp.int32, (2 * _B, _H), 0)
    giota = (lax.broadcasted_iota(jnp.int32, (2 * _B, _H), 1)
             + jnp.where(iota_row >= _B, _H, 0))
    iota_k = lax.broadcasted_iota(jnp.int32, (1, _K), 1)

    ms = scores
    vals = jnp.zeros((_B, _K), f32)
    idxs = jnp.zeros((_B, _K), jnp.int32)
    for i in range(_K):
        m2 = jnp.max(ms, axis=1, keepdims=True)
        m = jnp.maximum(m2[:_B], m2[_B:])
        mb = jnp.concatenate([m, m], axis=0)
        i2 = jnp.min(jnp.where(ms == mb, giota, _N),
                     axis=1, keepdims=True)
        idxv = jnp.minimum(i2[:_B], i2[_B:])
        ib = jnp.concatenate([idxv, idxv], axis=0)
        ms = jnp.where(giota == ib, _NEG, ms)
        vals = jnp.where(iota_k == i, m, vals)
        idxs = jnp.where(iota_k == i, idxv, idxs)

    iota_lane = lax.broadcasted_iota(jnp.int32, (_K, _N), 1)
    for b in range(_B):
        idxs_b = idxs[b:b + 1, :]
        idxs_c = jnp.transpose(idxs_b)
        oh = (iota_lane == idxs_c).astype(f32)
        g64_out[b] = jnp.dot(oh, ne[b], preferred_element_type=f32)
        ridx_out[b, :] = (idxs_b + b * _N).reshape(_K)
        for w in range(_NW // _B):
            cidx_out[b * (_NW // _B) + w, :] = idxs_b.reshape(_K)

    vals_out[...] = vals
    scores_out[...] = scores
    scorer_out[...] = scorer


def _sc_a2_body(ahat2, ridx_hbm, cidx_hbm, out_hbm,
                ridx_v, cidx_v, rows_v, a2_v, sem):
    wid = lax.axis_index("s") * _NC + lax.axis_index("c")
    base = wid * _RPW
    pltpu.sync_copy(ridx_hbm.at[wid], ridx_v)
    pltpu.sync_copy(cidx_hbm.at[wid], cidx_v)
    for ch in range(_RPW // _RC):
        pltpu.async_copy(ahat2.at[ridx_v.at[ch]], rows_v, sem).wait()
        for r in range(_RC):
            row_sel = jnp.full((16,), r, jnp.int32)
            for c in range(_K // 16):
                colv = cidx_v[pl.ds(c * 16, 16)]
                vals = plsc.load_gather(rows_v, [row_sel, colv])
                a2_v[ch * _RC + r, pl.ds(c * 16, 16)] = vals
    pltpu.sync_copy(a2_v, out_hbm.at[pl.ds(base, _RPW)])


def _phase1b_body(g64a, vals_in, scores_in, ht, winit, binit, gim,
                  wu, uu, bu_, wr, ur, br_, wh, uh, bh_,
                  t1_out, pol_out, ent_out):
    f32 = jnp.float32
    nt = (((1,), (1,)), ((), ()))
    scores = scores_in[...]
    vals = vals_in[...]

    smax8 = jnp.max(scores, axis=1, keepdims=True)
    smax = jnp.maximum(smax8[:_B], smax8[_B:])
    smaxb = jnp.concatenate([smax, smax], axis=0)
    sh = scores - smaxb
    ex = jnp.exp(sh)
    z8 = jnp.sum(ex, axis=1, keepdims=True)
    z = z8[:_B] + z8[_B:]
    sx8 = jnp.sum(ex * sh, axis=1, keepdims=True)
    sx = sx8[:_B] + sx8[_B:]
    logz = jnp.log(z)
    ent = logz - sx / z
    pol = jnp.mean(vals, axis=1, keepdims=True) - smax - logz

    htm = jnp.tanh(
        lax.dot_general(ht[...], winit[...], nt, preferred_element_type=f32)
        + binit[...])

    for b in range(_B):
        g64 = g64a[b]
        tw = jnp.tanh(jnp.transpose(vals[b:b + 1, :]))
        out64 = g64 * tw
        o64t = jnp.transpose(out64)
        z_topk = jnp.concatenate([o64t, o64t], axis=1)

        prev_q = lax.dot_general(htm[b:b + 1, :], gim[...],
                                 (((0,), (0,)), ((), ())),
                                 preferred_element_type=f32)

        upd = jax.nn.sigmoid(
            jnp.dot(wu[...], z_topk, preferred_element_type=f32)
            + jnp.dot(uu[...], prev_q, preferred_element_type=f32)
            + bu_[...])
        rst = jax.nn.sigmoid(
            jnp.dot(wr[...], z_topk, preferred_element_type=f32)
            + jnp.dot(ur[...], prev_q, preferred_element_type=f32)
            + br_[...])
        hcap = jnp.tanh(
            jnp.dot(wh[...], z_topk, preferred_element_type=f32)
            + jnp.dot(uh[...], rst * prev_q, preferred_element_type=f32)
            + bh_[...])
        gcn_w = (1.0 - upd) * prev_q + upd * hcap

        t1_out[b] = jnp.dot(g64, gcn_w, preferred_element_type=f32)

    pol_out[...] = jnp.transpose(pol).reshape(_B)
    ent_out[...] = jnp.transpose(ent).reshape(_B)


def _phase3_body(a2_in, t1, sw, ne_out):
    f32 = jnp.float32
    for b in range(_B):
        a2 = a2_in[b]
        colsum = jnp.sum(a2, axis=0, keepdims=True)
        di_row = lax.rsqrt(colsum)
        di_col = jnp.transpose(di_row)
        a2n = a2 * di_row * di_col

        ne2 = jax.nn.relu(jnp.dot(a2n, t1[b], preferred_element_type=f32))
        t2 = jnp.dot(ne2, sw[...], preferred_element_type=f32)
        ne3 = jax.nn.relu(jnp.dot(a2n, t2, preferred_element_type=f32))
        ne_out[b] = (ne2 + ne3) * 0.5


def kernel(Ahat, node_embs, mask, ht, W_map, b_map, Wu, Uu, bu, Wr, Ur, br,
           Wh, Uh, bh, GCN_init_mapping, W_init, b_init, static_weights):
    del mask
    f32 = jnp.float32

    vm = pl.BlockSpec(memory_space=pltpu.VMEM)
    g64a, vals, scores, ridx, cidx, scorer = pl.pallas_call(
        _phase1a_body,
        in_specs=[vm] * 4,
        out_specs=[vm] * 6,
        out_shape=[
            jax.ShapeDtypeStruct((_B, _K, _D), f32),
            jax.ShapeDtypeStruct((_B, _K), f32),
            jax.ShapeDtypeStruct((2 * _B, _N // 2), f32),
            jax.ShapeDtypeStruct((_B, _K), jnp.int32),
            jax.ShapeDtypeStruct((_NW, _K), jnp.int32),
            jax.ShapeDtypeStruct((_B, _D), f32),
        ],
    )(node_embs, ht, W_map, b_map.reshape(1, _D))

    sc_gather = functools.partial(
        pl.kernel,
        mesh=plsc.VectorSubcoreMesh(core_axis_name="c", subcore_axis_name="s"),
        compiler_params=pltpu.CompilerParams(needs_layout_passes=False),
        out_type=jax.ShapeDtypeStruct((_B * _K, _K), f32),
        scratch_types=[
            pltpu.VMEM((_RPW // _RC, _RC), jnp.int32),
            pltpu.VMEM((_K,), jnp.int32),
            pltpu.VMEM((_RC, _N), f32),
            pltpu.VMEM((_RPW, _K), f32),
            pltpu.SemaphoreType.DMA,
        ],
    )(_sc_a2_body)
    a2_flat = sc_gather(Ahat.reshape(_B * _N, _N),
                        ridx.reshape(_NW, _RPW // _RC, _RC), cidx)

    t1, pol, ent = pl.pallas_call(
        _phase1b_body,
        in_specs=[vm] * 16,
        out_specs=[vm] * 3,
        out_shape=[
            jax.ShapeDtypeStruct((_B, _K, _D), f32),
            jax.ShapeDtypeStruct((_B,), f32),
            jax.ShapeDtypeStruct((_B,), f32),
        ],
    )(g64a, vals, scores, ht, W_init, b_init.reshape(1, _D),
      GCN_init_mapping, Wu, Uu, bu, Wr, Ur, br, Wh, Uh, bh)

    ne = pl.pallas_call(
        _phase3_body,
        in_specs=[vm] * 3,
        out_specs=vm,
        out_shape=jax.ShapeDtypeStruct((_B, _K, _D), f32),
    )(a2_flat.reshape(_B, _K, _K), t1, static_weights)

    return ne, pol, scorer, ent


# speedup vs baseline: 1.1836x; 1.1836x over previous
import functools

import jax
import jax.numpy as jnp
from jax import lax
from jax.experimental import pallas as pl
from jax.experimental.pallas import tpu as pltpu
from jax.experimental.pallas import tpu_sc as plsc

_B, _N, _D, _R, _K = 4, 4096, 128, 256, 64
_NEG = -3.0e38
_NC, _NS = 2, 16
_NW = _NC * _NS
_RPW = (_B * _K) // _NW
_RC = 4


def _phase1a_body(ne, ht, wmap, bmap,
                  g64_out, vals_out, scores_out, ridx_out, cidx_out,
                  scorer_out):
    f32 = jnp.float32
    nt = (((1,), (1,)), ((), ()))

    htr = ht[...]
    scorer = jnp.tanh(
        lax.dot_general(htr, wmap[...], nt, preferred_element_type=f32)
        + bmap[...])
    snorm = jnp.sqrt(jnp.sum(scorer * scorer, axis=1, keepdims=True))

    _W = _N // 8
    sb = []
    for b in range(_B):
        chunks = [lax.dot_general(scorer[b:b + 1, :],
                                  ne[b, r * _W:(r + 1) * _W],
                                  nt, preferred_element_type=f32)
                  for r in range(8)]
        sb.append(jnp.concatenate(chunks, axis=0) / snorm[b, 0])

    giota = (lax.broadcasted_iota(jnp.int32, (8, _W), 1)
             + _W * lax.broadcasted_iota(jnp.int32, (8, _W), 0))
    iota_k = lax.broadcasted_iota(jnp.int32, (1, _K), 1)

    ms = list(sb)
    valsb = [jnp.zeros((1, _K), f32) for _ in range(_B)]
    idxsb = [jnp.zeros((1, _K), jnp.int32) for _ in range(_B)]
    for i in range(_K):
        for b in range(_B):
            m8 = jnp.max(ms[b], axis=1, keepdims=True)
            m = jnp.max(m8, axis=0, keepdims=True)
            i8 = jnp.min(jnp.where(ms[b] == m, giota, _N),
                         axis=1, keepdims=True)
            idx = jnp.min(i8, axis=0, keepdims=True)
            ms[b] = jnp.where(giota == idx, _NEG, ms[b])
            valsb[b] = jnp.where(iota_k == i, m, valsb[b])
            idxsb[b] = jnp.where(iota_k == i, idx, idxsb[b])
    vals = jnp.concatenate(valsb, axis=0)
    idxs = jnp.concatenate(idxsb, axis=0)
    scores = jnp.concatenate(sb, axis=0)

    iota_lane = lax.broadcasted_iota(jnp.int32, (_K, _N), 1)
    for b in range(_B):
        idxs_b = idxs[b:b + 1, :]
        idxs_c = jnp.transpose(idxs_b)
        oh = (iota_lane == idxs_c).astype(f32)
        g64_out[b] = jnp.dot(oh, ne[b], preferred_element_type=f32)
        ridx_out[b, :] = (idxs_b + b * _N).reshape(_K)
        for w in range(_NW // _B):
            cidx_out[b * (_NW // _B) + w, :] = idxs_b.reshape(_K)

    vals_out[...] = vals
    scores_out[...] = scores
    scorer_out[...] = scorer


def _sc_a2_body(ahat2, ridx_hbm, cidx_hbm, out_hbm,
                ridx_v, cidx_v, rows_v, a2_v, sem):
    wid = lax.axis_index("s") * _NC + lax.axis_index("c")
    base = wid * _RPW
    pltpu.sync_copy(ridx_hbm.at[wid], ridx_v)
    pltpu.sync_copy(cidx_hbm.at[wid], cidx_v)
    for ch in range(_RPW // _RC):
        pltpu.async_copy(ahat2.at[ridx_v.at[ch]], rows_v, sem).wait()
        for r in range(_RC):
            row_sel = jnp.full((16,), r, jnp.int32)
            for c in range(_K // 16):
                colv = cidx_v[pl.ds(c * 16, 16)]
                vals = plsc.load_gather(rows_v, [row_sel, colv])
                a2_v[ch * _RC + r, pl.ds(c * 16, 16)] = vals
    pltpu.sync_copy(a2_v, out_hbm.at[pl.ds(base, _RPW)])


def _phase1b_body(g64a, vals_in, scores_in, ht, winit, binit, gim,
                  wu, uu, bu_, wr, ur, br_, wh, uh, bh_,
                  t1_out, pol_out, ent_out):
    f32 = jnp.float32
    nt = (((1,), (1,)), ((), ()))
    scores = scores_in[...]
    vals = vals_in[...]

    polb, entb, smaxb = [], [], []
    for b in range(_B):
        s_b = scores[8 * b:8 * b + 8]
        smax = jnp.max(jnp.max(s_b, axis=1, keepdims=True),
                       axis=0, keepdims=True)
        sh = s_b - smax
        ex = jnp.exp(sh)
        z = jnp.sum(jnp.sum(ex, axis=1, keepdims=True),
                    axis=0, keepdims=True)
        sx = jnp.sum(jnp.sum(ex * sh, axis=1, keepdims=True),
                     axis=0, keepdims=True)
        logz = jnp.log(z)
        entb.append(logz - sx / z)
        polb.append(jnp.mean(vals[b:b + 1, :], axis=1, keepdims=True)
                    - smax - logz)
        smaxb.append(smax)
    pol = jnp.concatenate(polb, axis=0)
    ent = jnp.concatenate(entb, axis=0)

    htm = jnp.tanh(
        lax.dot_general(ht[...], winit[...], nt, preferred_element_type=f32)
        + binit[...])

    for b in range(_B):
        g64 = g64a[b]
        tw = jnp.tanh(jnp.transpose(vals[b:b + 1, :]))
        out64 = g64 * tw
        o64t = jnp.transpose(out64)
        z_topk = jnp.concatenate([o64t, o64t], axis=1)

        prev_q = lax.dot_general(htm[b:b + 1, :], gim[...],
                                 (((0,), (0,)), ((), ())),
                                 preferred_element_type=f32)

        upd = jax.nn.sigmoid(
            jnp.dot(wu[...], z_topk, preferred_element_type=f32)
            + jnp.dot(uu[...], prev_q, preferred_element_type=f32)
            + bu_[...])
        rst = jax.nn.sigmoid(
            jnp.dot(wr[...], z_topk, preferred_element_type=f32)
            + jnp.dot(ur[...], prev_q, preferred_element_type=f32)
            + br_[...])
        hcap = jnp.tanh(
            jnp.dot(wh[...], z_topk, preferred_element_type=f32)
            + jnp.dot(uh[...], rst * prev_q, preferred_element_type=f32)
            + bh_[...])
        gcn_w = (1.0 - upd) * prev_q + upd * hcap

        t1_out[b] = jnp.dot(g64, gcn_w, preferred_element_type=f32)

    pol_out[...] = jnp.transpose(pol).reshape(_B)
    ent_out[...] = jnp.transpose(ent).reshape(_B)


def _phase3_body(a2_in, t1, sw, ne_out):
    f32 = jnp.float32
    for b in range(_B):
        a2 = a2_in[b]
        colsum = jnp.sum(a2, axis=0, keepdims=True)
        di_row = lax.rsqrt(colsum)
        di_col = jnp.transpose(di_row)
        a2n = a2 * di_row * di_col

        ne2 = jax.nn.relu(jnp.dot(a2n, t1[b], preferred_element_type=f32))
        t2 = jnp.dot(ne2, sw[...], preferred_element_type=f32)
        ne3 = jax.nn.relu(jnp.dot(a2n, t2, preferred_element_type=f32))
        ne_out[b] = (ne2 + ne3) * 0.5


def kernel(Ahat, node_embs, mask, ht, W_map, b_map, Wu, Uu, bu, Wr, Ur, br,
           Wh, Uh, bh, GCN_init_mapping, W_init, b_init, static_weights):
    del mask
    f32 = jnp.float32

    vm = pl.BlockSpec(memory_space=pltpu.VMEM)
    g64a, vals, scores, ridx, cidx, scorer = pl.pallas_call(
        _phase1a_body,
        in_specs=[vm] * 4,
        out_specs=[vm] * 6,
        out_shape=[
            jax.ShapeDtypeStruct((_B, _K, _D), f32),
            jax.ShapeDtypeStruct((_B, _K), f32),
            jax.ShapeDtypeStruct((8 * _B, _N // 8), f32),
            jax.ShapeDtypeStruct((_B, _K), jnp.int32),
            jax.ShapeDtypeStruct((_NW, _K), jnp.int32),
            jax.ShapeDtypeStruct((_B, _D), f32),
        ],
    )(node_embs, ht, W_map, b_map.reshape(1, _D))

    sc_gather = functools.partial(
        pl.kernel,
        mesh=plsc.VectorSubcoreMesh(core_axis_name="c", subcore_axis_name="s"),
        compiler_params=pltpu.CompilerParams(needs_layout_passes=False),
        out_type=jax.ShapeDtypeStruct((_B * _K, _K), f32),
        scratch_types=[
            pltpu.VMEM((_RPW // _RC, _RC), jnp.int32),
            pltpu.VMEM((_K,), jnp.int32),
            pltpu.VMEM((_RC, _N), f32),
            pltpu.VMEM((_RPW, _K), f32),
            pltpu.SemaphoreType.DMA,
        ],
    )(_sc_a2_body)
    a2_flat = sc_gather(Ahat.reshape(_B * _N, _N),
                        ridx.reshape(_NW, _RPW // _RC, _RC), cidx)

    t1, pol, ent = pl.pallas_call(
        _phase1b_body,
        in_specs=[vm] * 16,
        out_specs=[vm] * 3,
        out_shape=[
            jax.ShapeDtypeStruct((_B, _K, _D), f32),
            jax.ShapeDtypeStruct((_B,), f32),
            jax.ShapeDtypeStruct((_B,), f32),
        ],
    )(g64a, vals, scores, ht, W_init, b_init.reshape(1, _D),
      GCN_init_mapping, Wu, Uu, bu, Wr, Ur, br, Wh, Uh, bh)

    ne = pl.pallas_call(
        _phase3_body,
        in_specs=[vm] * 3,
        out_specs=vm,
        out_shape=jax.ShapeDtypeStruct((_B, _K, _D), f32),
    )(a2_flat.reshape(_B, _K, _K), t1, static_weights)

    return ne, pol, scorer, ent


# speedup vs baseline: 1.5065x; 1.2728x over previous
import jax
import jax.numpy as jnp
from jax import lax
from jax.experimental import pallas as pl
from jax.experimental.pallas import tpu as pltpu

_B, _N, _D, _R, _K = 4, 4096, 128, 256, 64
_NEG = -3.0e38


def _sem_body(ahat, ne, ht, wmapT, bmap, wu, uu, bu_, wr, ur, br_, wh, uh,
              bh_, gim, winitT, binit, sw,
              ne_out, pol_out, scorer_out, ent_out,
              a1, sem):
    f32 = jnp.float32
    nt = (((1,), (1,)), ((), ()))

    htr = ht[...]
    scorer = jnp.tanh(
        jnp.dot(htr, wmapT[...], preferred_element_type=f32) + bmap[...])
    snorm = jnp.sqrt(jnp.sum(scorer * scorer, axis=1, keepdims=True))

    rows = []
    for b in range(_B):
        rows.append(lax.dot_general(scorer[b:b + 1, :], ne[b], nt,
                                    preferred_element_type=f32))
    scores = jnp.concatenate(rows, axis=0) / snorm

    iota_n = lax.broadcasted_iota(jnp.int32, (_B, _N), 1)
    iota_k = lax.broadcasted_iota(jnp.int32, (1, _K), 1)
    iota_b = lax.broadcasted_iota(jnp.int32, (_B, 1), 0)

    ms = scores
    vals = jnp.zeros((_B, _K), f32)
    idxs = jnp.zeros((_B, _K), jnp.int32)
    copies = []
    for i in range(_K):
        m = jnp.max(ms, axis=1, keepdims=True)
        idxv = jnp.min(jnp.where(ms == m, iota_n, _N),
                       axis=1, keepdims=True)
        ms = jnp.where(iota_n == idxv, _NEG, ms)
        vals = jnp.where(iota_k == i, m, vals)
        idxs = jnp.where(iota_k == i, idxv, idxs)
        for b in range(_B):
            idx_b = jnp.min(jnp.where(iota_b == b, idxv, _N))
            cp = pltpu.make_async_copy(ahat.at[b, idx_b],
                                       a1.at[b * _K + i], sem)
            cp.start()
            copies.append(cp)

    smax = jnp.max(scores, axis=1, keepdims=True)
    ex = jnp.exp(scores - smax)
    z = jnp.sum(ex, axis=1, keepdims=True)
    logz = jnp.log(z)
    ent = logz - jnp.sum(ex * (scores - smax), axis=1, keepdims=True) / z
    pol = jnp.mean(vals, axis=1, keepdims=True) - smax - logz

    htm = jnp.tanh(
        jnp.dot(htr, winitT[...], preferred_element_type=f32) + binit[...])

    iota_row = lax.broadcasted_iota(jnp.int32, (_N, _K), 0)
    iota_lane = lax.broadcasted_iota(jnp.int32, (_K, _N), 1)

    for cp in copies:
        cp.wait()

    for b in range(_B):
        idxs_b = idxs[b:b + 1, :]
        idxs_c = jnp.transpose(idxs_b)
        ohT = (iota_row == idxs_b).astype(f32)
        oh = (iota_lane == idxs_c).astype(f32)

        g64 = jnp.dot(oh, ne[b], preferred_element_type=f32)
        tw = jnp.tanh(jnp.transpose(vals[b:b + 1, :]))
        out64 = g64 * tw
        o64t = jnp.transpose(out64)
        z_topk = jnp.concatenate([o64t, o64t], axis=1)

        prev_q = lax.dot_general(htm[b:b + 1, :], gim[...],
                                 (((0,), (0,)), ((), ())),
                                 preferred_element_type=f32)

        upd = jax.nn.sigmoid(
            jnp.dot(wu[...], z_topk, preferred_element_type=f32)
            + jnp.dot(uu[...], prev_q, preferred_element_type=f32)
            + bu_[...])
        rst = jax.nn.sigmoid(
            jnp.dot(wr[...], z_topk, preferred_element_type=f32)
            + jnp.dot(ur[...], prev_q, preferred_element_type=f32)
            + br_[...])
        hcap = jnp.tanh(
            jnp.dot(wh[...], z_topk, preferred_element_type=f32)
            + jnp.dot(uh[...], rst * prev_q, preferred_element_type=f32)
            + bh_[...])
        gcn_w = (1.0 - upd) * prev_q + upd * hcap

        a1v = a1[b * _K:(b + 1) * _K, :]
        a2 = jnp.dot(a1v, ohT, preferred_element_type=f32)
        colsum = jnp.sum(a2, axis=0, keepdims=True)
        di_row = lax.rsqrt(colsum)
        di_col = jnp.transpose(di_row)
        a2n = a2 * di_row * di_col

        t1 = jnp.dot(g64, gcn_w, preferred_element_type=f32)
        ne2 = jax.nn.relu(jnp.dot(a2n, t1, preferred_element_type=f32))
        t2 = jnp.dot(ne2, sw[...], preferred_element_type=f32)
        ne3 = jax.nn.relu(jnp.dot(a2n, t2, preferred_element_type=f32))

        ne_out[b] = (ne2 + ne3) * 0.5

    pol_out[...] = pol
    scorer_out[...] = scorer
    ent_out[...] = ent


def kernel(Ahat, node_embs, mask, ht, W_map, b_map, Wu, Uu, bu, Wr, Ur, br,
           Wh, Uh, bh, GCN_init_mapping, W_init, b_init, static_weights):
    del mask
    f32 = jnp.float32

    vm = pl.BlockSpec(memory_space=pltpu.VMEM)
    ne, pol, scorer, ent = pl.pallas_call(
        _sem_body,
        in_specs=[pl.BlockSpec(memory_space=pl.ANY)] + [vm] * 17,
        out_specs=[vm, vm, vm, vm],
        out_shape=[
            jax.ShapeDtypeStruct((_B, _K, _D), f32),
            jax.ShapeDtypeStruct((_B, 1), f32),
            jax.ShapeDtypeStruct((_B, _D), f32),
            jax.ShapeDtypeStruct((_B, 1), f32),
        ],
        scratch_shapes=[
            pltpu.VMEM((_B * _K, _N), f32),
            pltpu.SemaphoreType.DMA,
        ],
    )(Ahat, node_embs, ht, W_map.T, b_map.reshape(1, _D), Wu, Uu, bu, Wr,
      Ur, br, Wh, Uh, bh, GCN_init_mapping, W_init.T, b_init.reshape(1, _D),
      static_weights)
    return ne, pol.reshape(_B), scorer, ent.reshape(_B)
